# baseline (speedup 1.0000x reference)
import jax
import jax.numpy as jnp
from jax import lax
from jax.experimental import pallas as pl
from jax.experimental.pallas import tpu as pltpu
from jax.experimental.pallas import tpu_sc as plsc

_N_BITS = 20
_MEM_SIZE = 2 ** _N_BITS
_BATCH = 16384

_NC = 2
_NS = 16
_LANES = 16
_NW = _NC * _NS
_BPW = _BATCH // _NW
_CHUNK = 128
_NCHUNK = _BPW // _CHUNK


_TC_BLK = 2048


def _tc_addr_kernel(bits_t_ref, idx_ref):
    acc = bits_t_ref[0, :]
    for j in range(1, _N_BITS):
        acc |= bits_t_ref[j, :] << j
    idx_ref[...] = acc.reshape(_TC_BLK // 128, 128)


def _sc_gather_kernel(idx_hbm, mem_hbm, out_hbm, idx_v, vals_v, sem, osem):
    wid = lax.axis_index("s") * _NC + lax.axis_index("c")
    base = wid * _BPW

    pltpu.sync_copy(idx_hbm.at[pl.ds(wid * _NCHUNK, _NCHUNK)], idx_v)

    copies = []
    for c in range(_NCHUNK):
        copies.append(
            pltpu.make_async_copy(
                mem_hbm.at[idx_v.at[c]],
                vals_v.at[pl.ds(c * _CHUNK, _CHUNK)],
                sem,
            )
        )
    for cp in copies:
        cp.start()
    outs = []
    for c, cp in enumerate(copies):
        cp.wait()
        ocp = pltpu.make_async_copy(
            vals_v.at[pl.ds(c * _CHUNK, _CHUNK)],
            out_hbm.at[pl.ds(base + c * _CHUNK, _CHUNK)],
            osem,
        )
        ocp.start()
        outs.append(ocp)
    for ocp in outs:
        ocp.wait()


@jax.jit
def kernel(bits, memory):
    bits_t = bits.T
    idx = pl.pallas_call(
        _tc_addr_kernel,
        grid=(_BATCH // _TC_BLK,),
        out_shape=jax.ShapeDtypeStruct((_BATCH // 128, 128), jnp.int32),
        in_specs=[pl.BlockSpec((_N_BITS, _TC_BLK), lambda i: (0, i))],
        out_specs=pl.BlockSpec((_TC_BLK // 128, 128), lambda i: (i, 0)),
    )(bits_t)

    mesh = plsc.VectorSubcoreMesh(
        core_axis_name="c", subcore_axis_name="s",
        num_cores=_NC, num_subcores=_NS,
    )
    out = pl.kernel(
        _sc_gather_kernel,
        out_type=jax.ShapeDtypeStruct((_BATCH,), jnp.int32),
        mesh=mesh,
        compiler_params=pltpu.CompilerParams(needs_layout_passes=False),
        scratch_types=[
            pltpu.VMEM((_NCHUNK, _CHUNK), jnp.int32),
            pltpu.VMEM((_BPW,), jnp.int32),
            pltpu.SemaphoreType.DMA,
            pltpu.SemaphoreType.DMA,
        ],
    )(idx, memory)
    return (out & 1).astype(jnp.bool_)


# speedup vs baseline: 1.1046x; 1.1046x over previous
import jax
import jax.numpy as jnp
from jax import lax
from jax.experimental import pallas as pl
from jax.experimental.pallas import tpu as pltpu
from jax.experimental.pallas import tpu_sc as plsc

_N_BITS = 20
_MEM_SIZE = 2 ** _N_BITS
_BATCH = 16384

_NC = 2
_NS = 16
_LANES = 16
_NW = _NC * _NS
_BPW = _BATCH // _NW
_CHUNK = 128
_NCHUNK = _BPW // _CHUNK


def _tc_addr_kernel(bits_t_ref, idx_ref):
    acc = bits_t_ref[0, :]
    for j in range(1, _N_BITS):
        acc |= bits_t_ref[j, :] << j
    idx_ref[...] = acc.reshape(_BATCH // 128, 128)


def _sc_gather_kernel(idx_hbm, mem_hbm, out_hbm, idx_v, vals_v, sem):
    wid = lax.axis_index("s") * _NC + lax.axis_index("c")

    pltpu.sync_copy(idx_hbm.at[pl.ds(wid * _NCHUNK, _NCHUNK)], idx_v)

    copies = []
    for c in range(_NCHUNK):
        copies.append(
            pltpu.make_async_copy(
                mem_hbm.at[idx_v.at[c]],
                vals_v.at[pl.ds(c * _CHUNK, _CHUNK)],
                sem,
            )
        )
    for cp in copies:
        cp.start()
    for cp in copies:
        cp.wait()

    pltpu.sync_copy(vals_v, out_hbm.at[pl.ds(wid * _BPW, _BPW)])


@jax.jit
def kernel(bits, memory):
    bits_t = bits.T
    idx = pl.pallas_call(
        _tc_addr_kernel,
        out_shape=jax.ShapeDtypeStruct((_BATCH // 128, 128), jnp.int32),
        in_specs=[pl.BlockSpec(memory_space=pltpu.VMEM)],
        out_specs=pl.BlockSpec(memory_space=pltpu.VMEM),
    )(bits_t)

    mesh = plsc.VectorSubcoreMesh(
        core_axis_name="c", subcore_axis_name="s",
        num_cores=_NC, num_subcores=_NS,
    )
    out = pl.kernel(
        _sc_gather_kernel,
        out_type=jax.ShapeDtypeStruct((_BATCH,), jnp.int32),
        mesh=mesh,
        compiler_params=pltpu.CompilerParams(needs_layout_passes=False),
        scratch_types=[
            pltpu.VMEM((_NCHUNK, _CHUNK), jnp.int32),
            pltpu.VMEM((_BPW,), jnp.int32),
            pltpu.SemaphoreType.DMA,
        ],
    )(idx, memory)
    return (out & 1).astype(jnp.bool_)


# speedup vs baseline: 1.1475x; 1.0388x over previous
import jax
import jax.numpy as jnp
from jax import lax
from jax.experimental import pallas as pl
from jax.experimental.pallas import tpu as pltpu
from jax.experimental.pallas import tpu_sc as plsc

_N_BITS = 20
_MEM_SIZE = 2 ** _N_BITS
_BATCH = 16384

_NC = 1
_NS = 16
_LANES = 16
_NW = _NC * _NS
_BPW = _BATCH // _NW
_CHUNK = 128
_NCHUNK = _BPW // _CHUNK


def _tc_addr_kernel(bits_t_ref, idx_ref):
    acc = bits_t_ref[0, :]
    for j in range(1, _N_BITS):
        acc |= bits_t_ref[j, :] << j
    idx_ref[...] = acc.reshape(_BATCH // 128, 128)


def _sc_gather_kernel(idx_hbm, mem_hbm, out_hbm, idx_v, vals_v, sem):
    wid = lax.axis_index("s") * _NC + lax.axis_index("c")

    pltpu.sync_copy(idx_hbm.at[pl.ds(wid * _NCHUNK, _NCHUNK)], idx_v)

    copies = []
    for c in range(_NCHUNK):
        copies.append(
            pltpu.make_async_copy(
                mem_hbm.at[idx_v.at[c]],
                vals_v.at[pl.ds(c * _CHUNK, _CHUNK)],
                sem,
            )
        )
    for cp in copies:
        cp.start()
    for cp in copies:
        cp.wait()

    pltpu.sync_copy(vals_v, out_hbm.at[pl.ds(wid * _BPW, _BPW)])


@jax.jit
def kernel(bits, memory):
    bits_t = bits.T
    idx = pl.pallas_call(
        _tc_addr_kernel,
        out_shape=jax.ShapeDtypeStruct((_BATCH // 128, 128), jnp.int32),
        in_specs=[pl.BlockSpec(memory_space=pltpu.VMEM)],
        out_specs=pl.BlockSpec(memory_space=pltpu.VMEM),
    )(bits_t)

    mesh = plsc.VectorSubcoreMesh(
        core_axis_name="c", subcore_axis_name="s",
        num_cores=_NC, num_subcores=_NS,
    )
    out = pl.kernel(
        _sc_gather_kernel,
        out_type=jax.ShapeDtypeStruct((_BATCH,), jnp.int32),
        mesh=mesh,
        compiler_params=pltpu.CompilerParams(needs_layout_passes=False),
        scratch_types=[
            pltpu.VMEM((_NCHUNK, _CHUNK), jnp.int32),
            pltpu.VMEM((_BPW,), jnp.int32),
            pltpu.SemaphoreType.DMA,
        ],
    )(idx, memory)
    return (out & 1).astype(jnp.bool_)
